# Initial kernel scaffold; baseline (speedup 1.0000x reference)
import jax
import jax.numpy as jnp
from jax.experimental import pallas as pl


def kernel(x, idx_b, idx_c, idx_s, idx_h):
    raise NotImplementedError("write your pallas kernel here")


# speedup vs baseline: 3.3362x; 3.3362x over previous
import numpy as np
import jax
import jax.numpy as jnp
from jax.experimental import pallas as pl

_H = _W = 512
_OUT = 224
_SB, _SC, _SS, _SH = 0.6, 0.8, 0.4, 0.2

_rng = np.random.default_rng(0)
_area = _rng.uniform(0.2, 1.0)
_SIDE = max(1, int(round(np.sqrt(_area) * min(_H, _W))))
_TOP = int(_rng.integers(0, _H - _SIDE + 1))
_LEFT = int(_rng.integers(0, _W - _SIDE + 1))
_APPLY_CJ = bool(_rng.uniform() < 0.8)
_FB = float(_rng.uniform(max(0.0, 1.0 - _SB), 1.0 + _SB)) if _SB > 0 else 1.0
_FC = float(_rng.uniform(max(0.0, 1.0 - _SC), 1.0 + _SC)) if _SC > 0 else 1.0
_FS = float(_rng.uniform(max(0.0, 1.0 - _SS), 1.0 + _SS)) if _SS > 0 else 1.0
_FH = float(_rng.uniform(-_SH, _SH)) if _SH > 0 else 0.0
_APPLY_GRAY = bool(_rng.uniform() < 0.2)
_APPLY_BLUR = bool(_rng.uniform() < 0.5)
_SIGMA = float(_rng.uniform(0.1, 2.0))
_APPLY_FLIP = bool(_rng.uniform() < 0.5)

_MEAN = (0.485, 0.456, 0.406)
_STD = (0.229, 0.224, 0.225)


def _resize_weights(in_size: int, out_size: int, offset: int, full: int) -> np.ndarray:
    scale = out_size / in_size
    kernel_scale = max(1.0 / scale, 1.0)
    sample_f = (np.arange(out_size, dtype=np.float64) + 0.5) / scale - 0.5
    x = np.abs(sample_f[:, None] - np.arange(in_size, dtype=np.float64)[None, :])
    w = np.maximum(0.0, 1.0 - x / kernel_scale)
    w = w / w.sum(axis=1, keepdims=True)
    out = np.zeros((out_size, full), dtype=np.float32)
    out[:, offset:offset + in_size] = w.astype(np.float32)
    return out


_RH = _resize_weights(_SIDE, _OUT, _TOP, _H)
_RWT = _resize_weights(_SIDE, _OUT, _LEFT, _W).T

if _APPLY_FLIP:
    _RWT = _RWT[:, ::-1].copy()

if _APPLY_BLUR:
    _r = 4
    _xs = np.arange(-_r, _r + 1, dtype=np.float32)
    _k = np.exp(-(_xs ** 2) / (2.0 * _SIGMA * _SIGMA))
    _k = _k / _k.sum()
    _BLUR = np.zeros((_OUT, _OUT), dtype=np.float32)
    for _i in range(_OUT):
        for _j, _kv in zip(range(_i - _r, _i + _r + 1), _k):
            if 0 <= _j < _OUT:
                _BLUR[_i, _j] += _kv
else:
    _BLUR = None


def _color_jitter_tiles(r, g, b):
    fb = jnp.float32(_FB)
    fc = jnp.float32(_FC)
    fs = jnp.float32(_FS)
    one = jnp.float32(1.0)
    r = jnp.clip(r * fb, 0.0, 1.0)
    g = jnp.clip(g * fb, 0.0, 1.0)
    b = jnp.clip(b * fb, 0.0, 1.0)
    gray = 0.2989 * r + 0.587 * g + 0.114 * b
    m = jnp.mean(gray)
    r = jnp.clip(fc * r + (one - fc) * m, 0.0, 1.0)
    g = jnp.clip(fc * g + (one - fc) * m, 0.0, 1.0)
    b = jnp.clip(fc * b + (one - fc) * m, 0.0, 1.0)
    r = jnp.clip(fs * r + (one - fs) * gray, 0.0, 1.0)
    g = jnp.clip(fs * g + (one - fs) * gray, 0.0, 1.0)
    b = jnp.clip(fs * b + (one - fs) * gray, 0.0, 1.0)
    maxc = jnp.maximum(jnp.maximum(r, g), b)
    minc = jnp.minimum(jnp.minimum(r, g), b)
    v = maxc
    delta = maxc - minc
    s = jnp.where(maxc > 1e-8, delta / jnp.maximum(maxc, 1e-8), 0.0)
    sd = jnp.where(delta < 1e-8, 1.0, delta)
    rc = (maxc - r) / sd
    gc = (maxc - g) / sd
    bc = (maxc - b) / sd
    h = jnp.where(maxc == r, bc - gc,
                  jnp.where(maxc == g, 2.0 + rc - bc, 4.0 + gc - rc))
    h = jnp.where(delta < 1e-8, 0.0, (h / 6.0) % 1.0)
    h = (h + jnp.float32(_FH)) % 1.0
    i = jnp.floor(h * 6.0)
    f = h * 6.0 - i
    p = v * (1.0 - s)
    q = v * (1.0 - f * s)
    t = v * (1.0 - (1.0 - f) * s)
    i = i.astype(jnp.int32) % 6

    def _pick(c0, c1, c2, c3, c4, c5):
        return jnp.where(i == 0, c0,
               jnp.where(i == 1, c1,
               jnp.where(i == 2, c2,
               jnp.where(i == 3, c3,
               jnp.where(i == 4, c4, c5)))))

    r = _pick(v, q, p, p, t, v)
    g = _pick(t, v, v, q, p, p)
    b = _pick(p, p, t, v, v, q)
    return (jnp.clip(r, 0.0, 1.0), jnp.clip(g, 0.0, 1.0), jnp.clip(b, 0.0, 1.0))


def _augment_body(x_ref, rh_ref, rwt_ref, o_ref):
    rh = rh_ref[...]
    rwt = rwt_ref[...]
    tiles = []
    for c in range(3):
        img = x_ref[0, c]
        t1 = jax.lax.dot(rh, img, precision=jax.lax.Precision.HIGHEST)
        t2 = jax.lax.dot(t1, rwt, precision=jax.lax.Precision.HIGHEST)
        tiles.append(jnp.clip(t2, 0.0, 1.0))
    r, g, b = tiles
    if _APPLY_CJ:
        r, g, b = _color_jitter_tiles(r, g, b)
    if _APPLY_GRAY:
        gray = 0.2989 * r + 0.587 * g + 0.114 * b
        r = g = b = gray
    if _BLUR is not None:
        bl = jnp.asarray(_BLUR)
        blt = bl.T
        r, g, b = (
            jax.lax.dot(jax.lax.dot(bl, t, precision=jax.lax.Precision.HIGHEST),
                        blt, precision=jax.lax.Precision.HIGHEST)
            for t in (r, g, b)
        )
    chans = (r, g, b)
    for c in range(3):
        val = (chans[c] - jnp.float32(_MEAN[c])) / jnp.float32(_STD[c])
        o_ref[0, 0, c] = val
        o_ref[1, 0, c] = val


def kernel(x, idx_b, idx_c, idx_s, idx_h):
    del idx_b, idx_c, idx_s, idx_h
    B = x.shape[0]
    rh = jnp.asarray(_RH)
    rwt = jnp.asarray(_RWT)
    out = pl.pallas_call(
        _augment_body,
        grid=(B,),
        in_specs=[
            pl.BlockSpec((1, 3, _H, _W), lambda i: (i, 0, 0, 0)),
            pl.BlockSpec((_OUT, _H), lambda i: (0, 0)),
            pl.BlockSpec((_W, _OUT), lambda i: (0, 0)),
        ],
        out_specs=pl.BlockSpec((2, 1, 3, _OUT, _OUT), lambda i: (0, i, 0, 0, 0)),
        out_shape=jax.ShapeDtypeStruct((2, B, 3, _OUT, _OUT), jnp.float32),
    )(x, rh, rwt)
    return out


# speedup vs baseline: 7.0757x; 2.1209x over previous
import numpy as np
import jax
import jax.numpy as jnp
from jax.experimental import pallas as pl

_H = _W = 512
_OUT = 224
_SB, _SC, _SS, _SH = 0.6, 0.8, 0.4, 0.2

_rng = np.random.default_rng(0)
_area = _rng.uniform(0.2, 1.0)
_SIDE = max(1, int(round(np.sqrt(_area) * min(_H, _W))))
_TOP = int(_rng.integers(0, _H - _SIDE + 1))
_LEFT = int(_rng.integers(0, _W - _SIDE + 1))
_APPLY_CJ = bool(_rng.uniform() < 0.8)
_FB = float(_rng.uniform(max(0.0, 1.0 - _SB), 1.0 + _SB)) if _SB > 0 else 1.0
_FC = float(_rng.uniform(max(0.0, 1.0 - _SC), 1.0 + _SC)) if _SC > 0 else 1.0
_FS = float(_rng.uniform(max(0.0, 1.0 - _SS), 1.0 + _SS)) if _SS > 0 else 1.0
_FH = float(_rng.uniform(-_SH, _SH)) if _SH > 0 else 0.0
_APPLY_GRAY = bool(_rng.uniform() < 0.2)
_APPLY_BLUR = bool(_rng.uniform() < 0.5)
_SIGMA = float(_rng.uniform(0.1, 2.0))
_APPLY_FLIP = bool(_rng.uniform() < 0.5)

_DOT_PRECISION = jax.lax.Precision.DEFAULT

_MEAN = (0.485, 0.456, 0.406)
_STD = (0.229, 0.224, 0.225)


def _resize_weights(in_size: int, out_size: int, offset: int, full: int) -> np.ndarray:
    scale = out_size / in_size
    kernel_scale = max(1.0 / scale, 1.0)
    sample_f = (np.arange(out_size, dtype=np.float64) + 0.5) / scale - 0.5
    x = np.abs(sample_f[:, None] - np.arange(in_size, dtype=np.float64)[None, :])
    w = np.maximum(0.0, 1.0 - x / kernel_scale)
    w = w / w.sum(axis=1, keepdims=True)
    out = np.zeros((out_size, full), dtype=np.float32)
    out[:, offset:offset + in_size] = w.astype(np.float32)
    return out


_RH = _resize_weights(_SIDE, _OUT, _TOP, _H)
_RWT = _resize_weights(_SIDE, _OUT, _LEFT, _W).T

if _APPLY_FLIP:
    _RWT = _RWT[:, ::-1].copy()

if _APPLY_BLUR:
    _r = 4
    _xs = np.arange(-_r, _r + 1, dtype=np.float32)
    _k = np.exp(-(_xs ** 2) / (2.0 * _SIGMA * _SIGMA))
    _k = _k / _k.sum()
    _BLUR = np.zeros((_OUT, _OUT), dtype=np.float32)
    for _i in range(_OUT):
        for _j, _kv in zip(range(_i - _r, _i + _r + 1), _k):
            if 0 <= _j < _OUT:
                _BLUR[_i, _j] += _kv
else:
    _BLUR = None


def _color_jitter_tiles(r, g, b):
    fb = jnp.float32(_FB)
    fc = jnp.float32(_FC)
    fs = jnp.float32(_FS)
    one = jnp.float32(1.0)
    r = jnp.clip(r * fb, 0.0, 1.0)
    g = jnp.clip(g * fb, 0.0, 1.0)
    b = jnp.clip(b * fb, 0.0, 1.0)
    gray = 0.2989 * r + 0.587 * g + 0.114 * b
    m = jnp.mean(gray)
    r = jnp.clip(fc * r + (one - fc) * m, 0.0, 1.0)
    g = jnp.clip(fc * g + (one - fc) * m, 0.0, 1.0)
    b = jnp.clip(fc * b + (one - fc) * m, 0.0, 1.0)
    r = jnp.clip(fs * r + (one - fs) * gray, 0.0, 1.0)
    g = jnp.clip(fs * g + (one - fs) * gray, 0.0, 1.0)
    b = jnp.clip(fs * b + (one - fs) * gray, 0.0, 1.0)
    maxc = jnp.maximum(jnp.maximum(r, g), b)
    minc = jnp.minimum(jnp.minimum(r, g), b)
    v = maxc
    delta = maxc - minc
    s = jnp.where(maxc > 1e-8, delta / jnp.maximum(maxc, 1e-8), 0.0)
    sd = jnp.where(delta < 1e-8, 1.0, delta)
    rc = (maxc - r) / sd
    gc = (maxc - g) / sd
    bc = (maxc - b) / sd
    h = jnp.where(maxc == r, bc - gc,
                  jnp.where(maxc == g, 2.0 + rc - bc, 4.0 + gc - rc))
    h = jnp.where(delta < 1e-8, 0.0, (h / 6.0) % 1.0)
    h = (h + jnp.float32(_FH)) % 1.0
    i = jnp.floor(h * 6.0)
    f = h * 6.0 - i
    p = v * (1.0 - s)
    q = v * (1.0 - f * s)
    t = v * (1.0 - (1.0 - f) * s)
    i = i.astype(jnp.int32) % 6

    def _pick(c0, c1, c2, c3, c4, c5):
        return jnp.where(i == 0, c0,
               jnp.where(i == 1, c1,
               jnp.where(i == 2, c2,
               jnp.where(i == 3, c3,
               jnp.where(i == 4, c4, c5)))))

    r = _pick(v, q, p, p, t, v)
    g = _pick(t, v, v, q, p, p)
    b = _pick(p, p, t, v, v, q)
    return (jnp.clip(r, 0.0, 1.0), jnp.clip(g, 0.0, 1.0), jnp.clip(b, 0.0, 1.0))


def _augment_body(x_ref, rh_ref, rwt_ref, o_ref):
    rh = rh_ref[...]
    rwt = rwt_ref[...]
    tiles = []
    for c in range(3):
        img = x_ref[0, c]
        t1 = jax.lax.dot(rh, img, precision=_DOT_PRECISION)
        t2 = jax.lax.dot(t1, rwt, precision=_DOT_PRECISION)
        tiles.append(jnp.clip(t2, 0.0, 1.0))
    r, g, b = tiles
    if _APPLY_CJ:
        r, g, b = _color_jitter_tiles(r, g, b)
    if _APPLY_GRAY:
        gray = 0.2989 * r + 0.587 * g + 0.114 * b
        r = g = b = gray
    if _BLUR is not None:
        bl = jnp.asarray(_BLUR)
        blt = bl.T
        r, g, b = (
            jax.lax.dot(jax.lax.dot(bl, t, precision=_DOT_PRECISION),
                        blt, precision=_DOT_PRECISION)
            for t in (r, g, b)
        )
    chans = (r, g, b)
    for c in range(3):
        val = (chans[c] - jnp.float32(_MEAN[c])) / jnp.float32(_STD[c])
        o_ref[0, 0, c] = val
        o_ref[1, 0, c] = val


def kernel(x, idx_b, idx_c, idx_s, idx_h):
    del idx_b, idx_c, idx_s, idx_h
    B = x.shape[0]
    rh = jnp.asarray(_RH)
    rwt = jnp.asarray(_RWT)
    out = pl.pallas_call(
        _augment_body,
        grid=(B,),
        in_specs=[
            pl.BlockSpec((1, 3, _H, _W), lambda i: (i, 0, 0, 0)),
            pl.BlockSpec((_OUT, _H), lambda i: (0, 0)),
            pl.BlockSpec((_W, _OUT), lambda i: (0, 0)),
        ],
        out_specs=pl.BlockSpec((2, 1, 3, _OUT, _OUT), lambda i: (0, i, 0, 0, 0)),
        out_shape=jax.ShapeDtypeStruct((2, B, 3, _OUT, _OUT), jnp.float32),
    )(x, rh, rwt)
    return out
